# baseline (speedup 1.0000x reference)
import functools

import numpy as np
import jax
import jax.numpy as jnp
from jax import lax
from jax.experimental import pallas as pl
from jax.experimental.pallas import tpu as pltpu
from jax.experimental.pallas import tpu_sc as plsc

NC, NS, L = 2, 16, 16
NW = NC * NS
ED = 10
GRP = 16

_PAT = np.array([[(v * L + l) // ED for l in range(L)] for v in range(ED)], np.int32)
_DIM = np.array([[(v * L + l) % ED for l in range(L)] for v in range(ED)], np.int32)


@functools.partial(jax.jit, static_argnames=("chunk",))
def _sc_embed(seq_flat, table, *, chunk):
    n = seq_flat.shape[0]
    nvocab = table.shape[0]
    per_w = n // NW
    n_chunks = per_w // chunk
    assert n == per_w * NW and per_w == n_chunks * chunk and chunk % GRP == 0

    mesh = plsc.VectorSubcoreMesh(
        core_axis_name="c", subcore_axis_name="s", num_cores=NC, num_subcores=NS
    )

    @functools.partial(
        pl.kernel,
        out_type=jax.ShapeDtypeStruct((n * ED,), jnp.float32),
        mesh=mesh,
        compiler_params=pltpu.CompilerParams(
            needs_layout_passes=False, disable_bounds_checks=True
        ),
        scratch_types=[
            pltpu.VMEM((nvocab, ED), jnp.float32),
            pltpu.VMEM((chunk,), jnp.int32),
            pltpu.VMEM((chunk * ED,), jnp.float32),
        ],
    )
    def run(seq_hbm, tab_hbm, out_hbm, tab_v, seq_v, out_v):
        wid = lax.axis_index("s") * NC + lax.axis_index("c")
        base = wid * per_w
        pltpu.sync_copy(tab_hbm, tab_v)
        lane = lax.iota(jnp.int32, L)
        pats = [lax.div(lane + v * L, ED) for v in range(ED)]
        dims = [lax.rem(lane + v * L, ED) for v in range(ED)]

        for c in range(n_chunks):
            off = base + c * chunk
            pltpu.sync_copy(seq_hbm.at[pl.ds(off, chunk)], seq_v)

            def body(g, carry):
                p0 = g * GRP
                for v in range(ED):
                    sv = plsc.load_gather(seq_v, [pats[v] + p0])
                    row = plsc.load_gather(tab_v, [sv, dims[v]])
                    out_v[pl.ds(g * (GRP * ED) + v * L, L)] = row
                return carry

            lax.fori_loop(0, chunk // GRP, body, 0)
            pltpu.sync_copy(out_v, out_hbm.at[pl.ds(off * ED, chunk * ED)])

    return run(seq_flat, table)


def kernel(sequence, table):
    b, s = sequence.shape
    v, d = table.shape
    assert d == ED
    seq_flat = sequence.reshape(-1).astype(jnp.int32)
    out_flat = _sc_embed(seq_flat, table.astype(jnp.float32), chunk=4096)
    return out_flat.reshape(b, s, d)


# speedup vs baseline: 4.0573x; 4.0573x over previous
import functools

import jax
import jax.numpy as jnp
from jax import lax
from jax.experimental import pallas as pl
from jax.experimental.pallas import tpu as pltpu
from jax.experimental.pallas import tpu_sc as plsc

NC, NS, L = 2, 16, 16
NW = NC * NS
ED = 10
NV = 25
B, S = 16384, 200
IT_PER_W = (B // 128) // NW
I_PER_W = IT_PER_W * 128
NPLANES = ED * (S // 8)


@jax.jit
def _sc_embed(seq_flat, tab_t):
    mesh = plsc.VectorSubcoreMesh(
        core_axis_name="c", subcore_axis_name="s", num_cores=NC, num_subcores=NS
    )

    @functools.partial(
        pl.kernel,
        out_type=jax.ShapeDtypeStruct((ED, S, B), jnp.float32),
        mesh=mesh,
        compiler_params=pltpu.CompilerParams(
            needs_layout_passes=False,
            disable_bounds_checks=True,
            use_tc_tiling_on_sc=True,
        ),
        scratch_types=[
            pltpu.VMEM((ED * NV,), jnp.float32),
            pltpu.VMEM((I_PER_W * S,), jnp.int32),
            pltpu.VMEM((8, IT_PER_W * 128), jnp.float32),
        ],
    )
    def run(seq_hbm, tab_hbm, out_hbm, tab_v, seq_v, buf_v):
        wid = lax.axis_index("s") * NC + lax.axis_index("c")
        pltpu.sync_copy(tab_hbm, tab_v)
        pltpu.sync_copy(seq_hbm.at[pl.ds(wid * (I_PER_W * S), I_PER_W * S)], seq_v)
        lane200 = lax.iota(jnp.int32, L) * S

        def plane(t, carry):
            d = t // (S // 8)
            jt = t % (S // 8)
            drow = d * NV

            def row(js, c2):
                jcol = jt * 8 + js
                for itl in range(IT_PER_W):
                    for il0 in range(8):
                        base = (itl * 128 + il0 * 16) * S + jcol
                        sv = plsc.load_gather(seq_v, [lane200 + base])
                        val = plsc.load_gather(tab_v, [sv + drow])
                        buf_v[js, pl.ds(itl * 128 + il0 * 16, L)] = val
                return c2

            lax.fori_loop(0, 8, row, 0)
            pltpu.sync_copy(
                buf_v,
                out_hbm.at[d, pl.ds(jt * 8, 8), pl.ds(wid * I_PER_W, I_PER_W)],
            )
            return carry

        lax.fori_loop(0, NPLANES, plane, 0)

    return run(seq_flat, tab_t)


def kernel(sequence, table):
    seq_flat = sequence.reshape(-1).astype(jnp.int32)
    tab_t = table.astype(jnp.float32).T.reshape(-1)
    out_t = _sc_embed(seq_flat, tab_t)
    return jnp.transpose(out_t, (2, 1, 0))


# speedup vs baseline: 9.3092x; 2.2944x over previous
import functools

import jax
import jax.numpy as jnp
from jax import lax
from jax.experimental import pallas as pl
from jax.experimental.pallas import tpu as pltpu
from jax.experimental.pallas import tpu_sc as plsc

NC, NS, L = 2, 16, 16
NW = NC * NS
ED = 10
NV = 25
B, S = 16384, 200
NJT = S // 8
IT_PER_W = (B // 128) // NW
I_PER_W = IT_PER_W * 128


@jax.jit
def _sc_embed(seq_flat, tab_t):
    mesh = plsc.VectorSubcoreMesh(
        core_axis_name="c", subcore_axis_name="s", num_cores=NC, num_subcores=NS
    )

    @functools.partial(
        pl.kernel,
        out_type=jax.ShapeDtypeStruct((ED, S, B), jnp.float32),
        mesh=mesh,
        compiler_params=pltpu.CompilerParams(
            needs_layout_passes=False,
            disable_bounds_checks=True,
            use_tc_tiling_on_sc=True,
        ),
        scratch_types=[
            pltpu.VMEM((ED * NV,), jnp.float32),
            pltpu.VMEM((I_PER_W * S,), jnp.int32),
            pltpu.VMEM((8, I_PER_W), jnp.int32),
            pltpu.VMEM((8, I_PER_W), jnp.float32),
            pltpu.VMEM((8, I_PER_W), jnp.float32),
            pltpu.SemaphoreType.DMA,
            pltpu.SemaphoreType.DMA,
        ],
    )
    def run(seq_hbm, tab_hbm, out_hbm, tab_v, seq_v, seqt_v, buf0, buf1, sem0, sem1):
        wid = lax.axis_index("s") * NC + lax.axis_index("c")
        pltpu.sync_copy(tab_hbm, tab_v)
        pltpu.sync_copy(seq_hbm.at[pl.ds(wid * (I_PER_W * S), I_PER_W * S)], seq_v)
        lane200 = lax.iota(jnp.int32, L) * S
        bufs = (buf0, buf1)
        sems = (sem0, sem1)
        i0 = wid * I_PER_W

        def plane(jt, carry):
            jcol0 = jt * 8

            def trow(js, c2):
                base = jcol0 + js
                for v16 in range(I_PER_W // L):
                    sv = plsc.load_gather(seq_v, [lane200 + (base + v16 * (L * S))])
                    seqt_v[js, pl.ds(v16 * L, L)] = sv
                return c2

            lax.fori_loop(0, 8, trow, 0)

            for d in range(ED):
                buf, sem = bufs[d % 2], sems[d % 2]
                dst = out_hbm.at[d, pl.ds(jcol0, 8), pl.ds(i0, I_PER_W)]

                if d < 2:
                    @pl.when(jt > 0)
                    def _():
                        pltpu.make_async_copy(buf, dst, sem).wait()
                else:
                    pltpu.make_async_copy(buf, dst, sem).wait()

                drow = d * NV

                def row(js, c2):
                    for v16 in range(I_PER_W // L):
                        sv = seqt_v[js, pl.ds(v16 * L, L)]
                        val = plsc.load_gather(tab_v, [sv + drow])
                        buf[js, pl.ds(v16 * L, L)] = val
                    return c2

                lax.fori_loop(0, 8, row, 0)
                pltpu.async_copy(buf, dst, sem)
            return carry

        lax.fori_loop(0, NJT, plane, 0)
        last = out_hbm.at[ED - 1, pl.ds((NJT - 1) * 8, 8), pl.ds(i0, I_PER_W)]
        pltpu.make_async_copy(buf0, last, sem0).wait()
        pltpu.make_async_copy(buf1, last, sem1).wait()

    return run(seq_flat, tab_t)


def kernel(sequence, table):
    seq_flat = sequence.reshape(-1).astype(jnp.int32)
    tab_t = table.astype(jnp.float32).T.reshape(-1)
    out_t = _sc_embed(seq_flat, tab_t)
    return jnp.transpose(out_t, (2, 1, 0))


# speedup vs baseline: 13.2649x; 1.4249x over previous
import functools

import jax
import jax.numpy as jnp
from jax import lax
from jax.experimental import pallas as pl
from jax.experimental.pallas import tpu as pltpu
from jax.experimental.pallas import tpu_sc as plsc

NC, NS, L = 2, 16, 16
NW = NC * NS
ED = 10
NV = 25
B, S = 16384, 200
NJT = S // 8
IT_PER_W = (B // 128) // NW
I_PER_W = IT_PER_W * 128


@jax.jit
def _sc_embed(seq_flat, tab_t):
    mesh = plsc.VectorSubcoreMesh(
        core_axis_name="c", subcore_axis_name="s", num_cores=NC, num_subcores=NS
    )

    @functools.partial(
        pl.kernel,
        out_type=jax.ShapeDtypeStruct((ED, S, B), jnp.float32),
        mesh=mesh,
        compiler_params=pltpu.CompilerParams(
            needs_layout_passes=False,
            disable_bounds_checks=True,
            use_tc_tiling_on_sc=True,
        ),
        scratch_types=[
            pltpu.VMEM((ED * NV,), jnp.float32),
            pltpu.VMEM((I_PER_W * S,), jnp.int32),
            pltpu.VMEM((8, I_PER_W), jnp.int32),
            pltpu.VMEM((8, I_PER_W), jnp.float32),
            pltpu.VMEM((8, I_PER_W), jnp.float32),
            pltpu.SemaphoreType.DMA,
            pltpu.SemaphoreType.DMA,
        ],
    )
    def run(seq_hbm, tab_hbm, out_hbm, tab_v, seq_v, seqt_v, buf0, buf1, sem0, sem1):
        wid = lax.axis_index("s") * NC + lax.axis_index("c")
        pltpu.sync_copy(tab_hbm, tab_v)
        pltpu.sync_copy(seq_hbm.at[pl.ds(wid * (I_PER_W * S), I_PER_W * S)], seq_v)
        lane200 = lax.iota(jnp.int32, L) * S
        bufs = (buf0, buf1)
        sems = (sem0, sem1)
        i0 = wid * I_PER_W

        def plane(jt, carry):
            jcol0 = jt * 8

            @plsc.parallel_loop(0, 8)
            def trow(js):
                base = jcol0 + js
                for v16 in range(I_PER_W // L):
                    sv = plsc.load_gather(seq_v, [lane200 + (base + v16 * (L * S))])
                    seqt_v[js, pl.ds(v16 * L, L)] = sv

            for d in range(ED):
                buf, sem = bufs[d % 2], sems[d % 2]
                dst = out_hbm.at[d, pl.ds(jcol0, 8), pl.ds(i0, I_PER_W)]

                if d < 2:
                    @pl.when(jt > 0)
                    def _():
                        pltpu.make_async_copy(buf, dst, sem).wait()
                else:
                    pltpu.make_async_copy(buf, dst, sem).wait()

                drow = d * NV

                @plsc.parallel_loop(0, 8)
                def row(js):
                    for v16 in range(I_PER_W // L):
                        sv = seqt_v[js, pl.ds(v16 * L, L)]
                        val = plsc.load_gather(tab_v, [sv + drow])
                        buf[js, pl.ds(v16 * L, L)] = val

                pltpu.async_copy(buf, dst, sem)
            return carry

        lax.fori_loop(0, NJT, plane, 0)
        last = out_hbm.at[ED - 1, pl.ds((NJT - 1) * 8, 8), pl.ds(i0, I_PER_W)]
        pltpu.make_async_copy(buf0, last, sem0).wait()
        pltpu.make_async_copy(buf1, last, sem1).wait()

    return run(seq_flat, tab_t)


def kernel(sequence, table):
    seq_flat = sequence.reshape(-1).astype(jnp.int32)
    tab_t = table.astype(jnp.float32).T.reshape(-1)
    out_t = _sc_embed(seq_flat, tab_t)
    return jnp.transpose(out_t, (2, 1, 0))


# speedup vs baseline: 17.9125x; 1.3504x over previous
import functools

import jax
import jax.numpy as jnp
from jax import lax
from jax.experimental import pallas as pl
from jax.experimental.pallas import tpu as pltpu
from jax.experimental.pallas import tpu_sc as plsc

NC, NS, L = 2, 16, 16
NW = NC * NS
ED = 10
NV = 25
B, S = 16384, 200
NJT = S // 8
IT_PER_W = (B // 128) // NW
I_PER_W = IT_PER_W * 128


@jax.jit
def _sc_embed(seq_flat, tab_t):
    mesh = plsc.VectorSubcoreMesh(
        core_axis_name="c", subcore_axis_name="s", num_cores=NC, num_subcores=NS
    )

    @functools.partial(
        pl.kernel,
        out_type=jax.ShapeDtypeStruct((ED, S, B), jnp.float32),
        mesh=mesh,
        compiler_params=pltpu.CompilerParams(
            needs_layout_passes=False,
            disable_bounds_checks=True,
            use_tc_tiling_on_sc=True,
        ),
        scratch_types=[
            pltpu.VMEM((ED * NV,), jnp.float32),
            pltpu.VMEM((I_PER_W * S,), jnp.int32),
            pltpu.VMEM((8, I_PER_W), jnp.int32),
            pltpu.VMEM((8, I_PER_W), jnp.float32),
            pltpu.VMEM((8, I_PER_W), jnp.float32),
            pltpu.VMEM((8, I_PER_W), jnp.float32),
            pltpu.VMEM((8, I_PER_W), jnp.float32),
            pltpu.SemaphoreType.DMA,
            pltpu.SemaphoreType.DMA,
            pltpu.SemaphoreType.DMA,
            pltpu.SemaphoreType.DMA,
        ],
    )
    def run(seq_hbm, tab_hbm, out_hbm, tab_v, seq_v, seqt_v,
            buf00, buf01, buf10, buf11, sem00, sem01, sem10, sem11):
        wid = lax.axis_index("s") * NC + lax.axis_index("c")
        pltpu.sync_copy(tab_hbm, tab_v)
        pltpu.sync_copy(seq_hbm.at[pl.ds(wid * (I_PER_W * S), I_PER_W * S)], seq_v)
        lane200 = lax.iota(jnp.int32, L) * S
        bufs = ((buf00, buf01), (buf10, buf11))
        sems = ((sem00, sem01), (sem10, sem11))
        i0 = wid * I_PER_W

        def plane(jt, carry):
            jcol0 = jt * 8

            @plsc.parallel_loop(0, 8)
            def trow(js):
                base = jcol0 + js
                for v16 in range(I_PER_W // L):
                    sv = plsc.load_gather(seq_v, [lane200 + (base + v16 * (L * S))])
                    seqt_v[js, pl.ds(v16 * L, L)] = sv

            for k in range(ED // 2):
                d0, d1 = 2 * k, 2 * k + 1
                (b0, b1), (s0, s1) = bufs[k % 2], sems[k % 2]
                dst0 = out_hbm.at[d0, pl.ds(jcol0, 8), pl.ds(i0, I_PER_W)]
                dst1 = out_hbm.at[d1, pl.ds(jcol0, 8), pl.ds(i0, I_PER_W)]

                if k < 2:
                    @pl.when(jt > 0)
                    def _():
                        pltpu.make_async_copy(b0, dst0, s0).wait()
                        pltpu.make_async_copy(b1, dst1, s1).wait()
                else:
                    pltpu.make_async_copy(b0, dst0, s0).wait()
                    pltpu.make_async_copy(b1, dst1, s1).wait()

                @plsc.parallel_loop(0, 8)
                def row(js):
                    for v16 in range(I_PER_W // L):
                        sv = seqt_v[js, pl.ds(v16 * L, L)]
                        val0 = plsc.load_gather(tab_v, [sv + d0 * NV])
                        val1 = plsc.load_gather(tab_v, [sv + d1 * NV])
                        b0[js, pl.ds(v16 * L, L)] = val0
                        b1[js, pl.ds(v16 * L, L)] = val1

                pltpu.async_copy(b0, dst0, s0)
                pltpu.async_copy(b1, dst1, s1)
            return carry

        lax.fori_loop(0, NJT, plane, 0)
        last = out_hbm.at[ED - 1, pl.ds((NJT - 1) * 8, 8), pl.ds(i0, I_PER_W)]
        pltpu.make_async_copy(buf00, last, sem00).wait()
        pltpu.make_async_copy(buf01, last, sem01).wait()
        pltpu.make_async_copy(buf10, last, sem10).wait()
        pltpu.make_async_copy(buf11, last, sem11).wait()

    return run(seq_flat, tab_t)


def kernel(sequence, table):
    seq_flat = sequence.reshape(-1).astype(jnp.int32)
    tab_t = table.astype(jnp.float32).T.reshape(-1)
    out_t = _sc_embed(seq_flat, tab_t)
    return jnp.transpose(out_t, (2, 1, 0))
